# baseline (speedup 1.0000x reference)
import functools
import jax
import jax.numpy as jnp
from jax import lax
from jax.experimental import pallas as pl
from jax.experimental.pallas import tpu as pltpu
from jax.experimental.pallas import tpu_sc as plsc

B = 16384
K = 200
NC = 2
NS = 16
NW = NC * NS
B_SC = 8192
B_TC = B - B_SC
RPW = B_SC // NW
NG = RPW // 16
NB_TC = B_TC // 128
UNROLL = 8
NCHUNK = K // UNROLL
EPS = 1e-7
LN2 = 0.6931471805599453
MASK23 = 0x007FFFFF
ONEBITS = 0x3F800000
C0 = 2.885390081777927
C1 = 0.961796693925976
C2 = 0.5770780163555854
C3 = 0.41219858311113246
C4 = 0.32059889797532526


def _log2_mant(m):
    u = (m - 1.0) / (m + 1.0)
    u2 = u * u
    return u * (C0 + u2 * (C1 + u2 * (C2 + u2 * (C3 + u2 * C4))))


def _sc_body(predt_hbm, edges_hbm, dur_hbm, ev_hbm, out_hbm,
             pred_v, edges_v, dur_v, ev_v, stage_v, sem0, sem1):
    wid = lax.axis_index("s") * NC + lax.axis_index("c")
    base = wid * RPW
    half = RPW // 2
    cp0 = pltpu.async_copy(predt_hbm.at[:, pl.ds(base, half)],
                           pred_v.at[:, pl.ds(0, half)], sem0)
    cp1 = pltpu.async_copy(predt_hbm.at[:, pl.ds(base + half, half)],
                           pred_v.at[:, pl.ds(half, half)], sem1)
    pltpu.sync_copy(edges_hbm, edges_v)
    pltpu.sync_copy(dur_hbm.at[pl.ds(base, RPW)], dur_v)
    pltpu.sync_copy(ev_hbm.at[pl.ds(base, RPW)], ev_v)

    lanes = lax.iota(jnp.int32, 16)
    inv_step = edges_v[pl.ds(208, 16)]

    def group_body(g, carry):
        acc_s, acc_h, dec_f = carry
        go = g * 16
        d = dur_v[pl.ds(go, 16)]
        evv = ev_v[pl.ds(go, 16)]
        is_ev = evv != 0

        a = d * inv_step
        c = a.astype(jnp.int32)
        bb = jnp.clip(c - 1, 0, K - 3)
        p = bb
        for t in range(4):
            ec = plsc.load_gather(edges_v, [jnp.minimum(bb + t, K)])
            p = p + jnp.where(ec < d, 1, 0).astype(jnp.int32)
        idx = jnp.clip(p - 1, 0, K - 1)

        def chunk_body(jj, ch):
            (e_s, m_s, e_h, m_h, prev_x, s_prev, dmin) = ch
            j0 = jj * UNROLL
            for dj in range(UNROLL):
                j = j0 + dj
                x = pred_v[j, pl.ds(go, 16)]
                dmin = jnp.minimum(dmin, prev_x - x)
                prev_x = x
                m_lt = j < idx
                t_h = jnp.clip(1.0 - x, EPS, 1.0 - EPS)
                t_h = jnp.where(m_lt, t_h, 1.0)
                tb = plsc.bitcast(t_h, jnp.int32)
                e_h = e_h + (tb >> 23)
                m_h = m_h * plsc.bitcast((tb & MASK23) | ONEBITS, jnp.float32)
                s = jnp.maximum(x, EPS)
                t_s = jnp.minimum(s / s_prev, 1.0 - EPS)
                s_prev = s
                t_s = jnp.where(m_lt, t_s, 1.0)
                tb = plsc.bitcast(t_s, jnp.int32)
                e_s = e_s + (tb >> 23)
                m_s = m_s * plsc.bitcast((tb & MASK23) | ONEBITS, jnp.float32)
            mb = plsc.bitcast(m_s, jnp.int32)
            e_s = e_s + (mb >> 23)
            m_s = plsc.bitcast((mb & MASK23) | ONEBITS, jnp.float32)
            mb = plsc.bitcast(m_h, jnp.int32)
            e_h = e_h + (mb >> 23)
            m_h = plsc.bitcast((mb & MASK23) | ONEBITS, jnp.float32)
            return (e_s, m_s, e_h, m_h, prev_x, s_prev, dmin)

        zi = lanes * 0
        zf = zi.astype(jnp.float32)
        init = (zi, zf + 1.0, zi, zf + 1.0, zf + 3e38, zf + 1.0, zf + 3e38)
        (e_s, m_s, e_h, m_h, _, _, dmin) = lax.fori_loop(
            0, NCHUNK, chunk_body, init)
        dec_f = jnp.minimum(dec_f, jnp.where(dmin >= -1e-6, 1.0, 0.0))

        cols = go + lanes
        x_at = plsc.load_gather(pred_v, [idx, cols])
        x_pv = plsc.load_gather(pred_v, [jnp.maximum(idx - 1, 0), cols])
        h_h_at = jnp.clip(x_at, EPS, 1.0 - EPS)
        s_at = jnp.clip(x_at, EPS, 1.0)
        s_pv = jnp.where(idx == 0, 1.0, jnp.clip(x_pv, EPS, 1.0))
        h_s_at = jnp.clip(1.0 - s_at / s_pv, EPS, 1.0 - EPS)

        ebias = 127 * (K + NCHUNK + 2)
        tail_s = jnp.where(is_ev, h_s_at, 1.0 - h_s_at)
        tb = plsc.bitcast(tail_s, jnp.int32)
        e_s = e_s + (tb >> 23)
        mm = m_s * plsc.bitcast((tb & MASK23) | ONEBITS, jnp.float32)
        mb = plsc.bitcast(mm, jnp.int32)
        e_s = e_s + (mb >> 23)
        m_c = plsc.bitcast((mb & MASK23) | ONEBITS, jnp.float32)
        ll_s = ((e_s - ebias).astype(jnp.float32) + _log2_mant(m_c)) * LN2
        fin_s = (ll_s > -1e30) & (ll_s < 1e30)
        acc_s = acc_s + jnp.where(fin_s, ll_s, -1e6)

        tail_h = jnp.where(is_ev, h_h_at, 1.0 - h_h_at)
        tb = plsc.bitcast(tail_h, jnp.int32)
        e_h = e_h + (tb >> 23)
        mm = m_h * plsc.bitcast((tb & MASK23) | ONEBITS, jnp.float32)
        mb = plsc.bitcast(mm, jnp.int32)
        e_h = e_h + (mb >> 23)
        m_c = plsc.bitcast((mb & MASK23) | ONEBITS, jnp.float32)
        ll_h = ((e_h - ebias).astype(jnp.float32) + _log2_mant(m_c)) * LN2
        fin_h = (ll_h > -1e30) & (ll_h < 1e30)
        acc_h = acc_h + jnp.where(fin_h, ll_h, -1e6)

        return (acc_s, acc_h, dec_f)

    zf = lanes.astype(jnp.float32) * 0.0
    carry = (zf, zf, zf + 1.0)
    cp0.wait()
    carry = lax.fori_loop(0, NG // 2, group_body, carry)
    cp1.wait()
    carry = lax.fori_loop(NG // 2, NG, group_body, carry)
    acc_s, acc_h, dec_f = carry

    stage_v[pl.ds(0, 16)] = acc_s
    stage_v[pl.ds(16, 16)] = acc_h
    stage_v[pl.ds(32, 16)] = dec_f
    stage_v[pl.ds(48, 16)] = dec_f
    pltpu.sync_copy(stage_v, out_hbm.at[wid])


def _tc_body(edges_ref, pred_ref, dur_ref, ev_ref, out_ref):
    d = dur_ref[0]
    is_ev = ev_ref[0] != 0
    db = jnp.broadcast_to(d, (8, 128))

    cnt = jnp.zeros((8, 128), jnp.int32)
    for c in range(NCHUNK):
        ec = jnp.broadcast_to(edges_ref[:, c:c + 1], (8, 128))
        cnt = cnt + jnp.where(ec < db, 1, 0).astype(jnp.int32)
    p = jnp.sum(cnt, axis=0, keepdims=True)
    idx = jnp.clip(p - 1, 0, K - 1)
    idx_b = jnp.broadcast_to(idx, (8, 128))

    ji0 = lax.broadcasted_iota(jnp.int32, (8, 128), 0)
    ev_b = jnp.broadcast_to(is_ev, (8, 128))

    zf8 = jnp.zeros((8, 128), jnp.float32)
    zi8 = jnp.zeros((8, 128), jnp.int32)
    at_x, at_pv = zf8, zf8
    m_s8, m_h8 = zf8 + 1.0, zf8 + 1.0
    e_s8, e_h8 = zi8, zi8
    dmin = zf8 + 3e38
    carry = jnp.ones((1, 128), jnp.float32)
    for c in range(NCHUNK):
        x8 = pred_ref[pl.ds(c * 8, 8), :]
        prev8 = pltpu.roll(x8, 1, 0)
        prev8 = jnp.where(ji0 == 0, jnp.broadcast_to(carry, (8, 128)), prev8)
        carry = x8[7:8, :]
        dmin = jnp.minimum(dmin, prev8 - x8)
        ji = ji0 + (c * 8)
        m_lt = ji < idx_b
        m_eq = ji == idx_b
        at_x = at_x + jnp.where(m_eq, x8, 0.0)
        at_pv = at_pv + jnp.where(m_eq, prev8, 0.0)
        t_h = jnp.clip(1.0 - x8, EPS, 1.0 - EPS)
        t_h = jnp.where(m_lt, t_h, 1.0)
        tb = lax.bitcast_convert_type(t_h, jnp.int32)
        e_h8 = e_h8 + (tb >> 23)
        m_h8 = m_h8 * lax.bitcast_convert_type(
            (tb & MASK23) | ONEBITS, jnp.float32)
        s = jnp.maximum(x8, EPS)
        sp = jnp.maximum(prev8, EPS)
        t_s = jnp.minimum(s / sp, 1.0 - EPS)
        t_s = jnp.where(m_lt, t_s, 1.0)
        tb = lax.bitcast_convert_type(t_s, jnp.int32)
        e_s8 = e_s8 + (tb >> 23)
        m_s8 = m_s8 * lax.bitcast_convert_type(
            (tb & MASK23) | ONEBITS, jnp.float32)

    x_at = jnp.sum(at_x, axis=0, keepdims=True)
    x_pv = jnp.sum(at_pv, axis=0, keepdims=True)
    dminv = jnp.min(dmin, axis=0, keepdims=True)
    decv = jnp.where(dminv >= -1e-6, 1.0, 0.0)

    h_h_at = jnp.clip(x_at, EPS, 1.0 - EPS)
    tail_h = jnp.where(is_ev, h_h_at, 1.0 - h_h_at)
    s_at = jnp.clip(x_at, EPS, 1.0)
    s_pv = jnp.clip(x_pv, EPS, 1.0)
    h_s_at = jnp.clip(1.0 - s_at / s_pv, EPS, 1.0 - EPS)
    tail_s = jnp.where(is_ev, h_s_at, 1.0 - h_s_at)

    ebias = 127 * (K + 8 + 3)

    def branch_ll(e8, m8, tail):
        mb = lax.bitcast_convert_type(m8, jnp.int32)
        e8 = e8 + (mb >> 23)
        mant8 = lax.bitcast_convert_type((mb & MASK23) | ONEBITS, jnp.float32)
        r = mant8 * pltpu.roll(mant8, 4, 0)
        r = r * pltpu.roll(r, 2, 0)
        r = r * pltpu.roll(r, 1, 0)
        esum = jnp.sum(e8, axis=0, keepdims=True)
        r0 = r[0:1, :]
        rb = lax.bitcast_convert_type(r0, jnp.int32)
        esum = esum + (rb >> 23)
        mc = lax.bitcast_convert_type((rb & MASK23) | ONEBITS, jnp.float32)
        tb2 = lax.bitcast_convert_type(tail, jnp.int32)
        esum = esum + (tb2 >> 23)
        mm = mc * lax.bitcast_convert_type(
            (tb2 & MASK23) | ONEBITS, jnp.float32)
        mb2 = lax.bitcast_convert_type(mm, jnp.int32)
        esum = esum + (mb2 >> 23)
        m_f = lax.bitcast_convert_type((mb2 & MASK23) | ONEBITS, jnp.float32)
        return ((esum - ebias).astype(jnp.float32) + _log2_mant(m_f)) * LN2

    ll_s = branch_ll(e_s8, m_s8, tail_s)
    ll_h = branch_ll(e_h8, m_h8, tail_h)
    ll_s = jnp.where((ll_s > -1e30) & (ll_s < 1e30), ll_s, -1e6)
    ll_h = jnp.where((ll_h > -1e30) & (ll_h < 1e30), ll_h, -1e6)

    z = jnp.zeros((1, 128), jnp.float32)
    out_ref[0] = jnp.concatenate(
        [ll_s, ll_h, decv, z, z, z, z, z], axis=0)


@jax.jit
def kernel(pred_prob, true_time, true_event):
    pred = pred_prob.astype(jnp.float32)
    dur = true_time.astype(jnp.float32).reshape(-1)
    ev = true_event.reshape(-1).astype(jnp.int32)
    max_t = jnp.clip(jnp.max(dur), 1e-6, None)
    edges = jnp.linspace(0.0, max_t, K + 1).astype(jnp.float32)
    edges_pad = jnp.zeros((224,), jnp.float32)
    edges_pad = edges_pad.at[:K + 1].set(edges)
    edges_pad = edges_pad.at[208:].set(jnp.float32(K) / max_t)
    edges_tc = jnp.zeros((8, 128), jnp.float32)
    edges_tc = edges_tc.at[:, :NCHUNK].set(edges[:K].reshape(NCHUNK, 8).T)

    predt = pred.T
    dur3 = dur.reshape(B // 128, 1, 128)
    ev3 = ev.reshape(B // 128, 1, 128)

    mesh = plsc.VectorSubcoreMesh(core_axis_name="c", subcore_axis_name="s",
                                  num_cores=NC, num_subcores=NS)
    run_sc = pl.kernel(
        _sc_body,
        out_type=jax.ShapeDtypeStruct((NW, 64), jnp.float32),
        mesh=mesh,
        compiler_params=pltpu.CompilerParams(needs_layout_passes=False),
        scratch_types=[
            pltpu.VMEM((K, RPW), jnp.float32),
            pltpu.VMEM((224,), jnp.float32),
            pltpu.VMEM((RPW,), jnp.float32),
            pltpu.VMEM((RPW,), jnp.int32),
            pltpu.VMEM((64,), jnp.float32),
            pltpu.SemaphoreType.DMA,
            pltpu.SemaphoreType.DMA,
        ],
    )
    parts_sc = run_sc(predt, edges_pad, dur, ev)

    blk0 = B_SC // 128
    parts_tc = pl.pallas_call(
        _tc_body,
        grid=(NB_TC,),
        in_specs=[
            pl.BlockSpec((8, 128), lambda i: (0, 0)),
            pl.BlockSpec((K, 128), lambda i: (0, i + blk0)),
            pl.BlockSpec((1, 1, 128), lambda i: (i + blk0, 0, 0)),
            pl.BlockSpec((1, 1, 128), lambda i: (i + blk0, 0, 0)),
        ],
        out_specs=pl.BlockSpec((1, 8, 128), lambda i: (i, 0, 0)),
        out_shape=jax.ShapeDtypeStruct((NB_TC, 8, 128), jnp.float32),
    )(edges_tc, predt, dur3, ev3)

    sum_s = jnp.sum(parts_sc[:, 0:16]) + jnp.sum(parts_tc[:, 0, :])
    sum_h = jnp.sum(parts_sc[:, 16:32]) + jnp.sum(parts_tc[:, 1, :])
    dec = jnp.minimum(jnp.min(parts_sc[:, 32:48]), jnp.min(parts_tc[:, 2, :]))
    cond = dec > 0.5
    return -jnp.where(cond, sum_s, sum_h) / B


# speedup vs baseline: 1.4404x; 1.4404x over previous
import functools
import jax
import jax.numpy as jnp
from jax import lax
from jax.experimental import pallas as pl
from jax.experimental.pallas import tpu as pltpu
from jax.experimental.pallas import tpu_sc as plsc

B = 16384
K = 200
NC = 2
NS = 16
NW = NC * NS
B_SC = 8192
B_TC = B - B_SC
RPW = B_SC // NW
NG = RPW // 16
LTC = 1024
NB_TC = B_TC // LTC
UNROLL = 8
NCHUNK = K // UNROLL
EPS = 1e-7
LN2 = 0.6931471805599453
MASK23 = 0x007FFFFF
ONEBITS = 0x3F800000
C0 = 2.885390081777927
C1 = 0.961796693925976
C2 = 0.5770780163555854
C3 = 0.41219858311113246
C4 = 0.32059889797532526


def _log2_mant(m):
    u = (m - 1.0) / (m + 1.0)
    u2 = u * u
    return u * (C0 + u2 * (C1 + u2 * (C2 + u2 * (C3 + u2 * C4))))


def _sc_body(predt_hbm, edges_hbm, dur_hbm, ev_hbm, out_hbm,
             pred_v, edges_v, dur_v, ev_v, stage_v, sem0, sem1):
    wid = lax.axis_index("s") * NC + lax.axis_index("c")
    base = wid * RPW
    half = RPW // 2
    cp0 = pltpu.async_copy(predt_hbm.at[:, pl.ds(base, half)],
                           pred_v.at[:, pl.ds(0, half)], sem0)
    cp1 = pltpu.async_copy(predt_hbm.at[:, pl.ds(base + half, half)],
                           pred_v.at[:, pl.ds(half, half)], sem1)
    pltpu.sync_copy(edges_hbm, edges_v)
    pltpu.sync_copy(dur_hbm.at[pl.ds(base, RPW)], dur_v)
    pltpu.sync_copy(ev_hbm.at[pl.ds(base, RPW)], ev_v)

    lanes = lax.iota(jnp.int32, 16)
    inv_step = edges_v[pl.ds(208, 16)]

    def group_body(g, carry):
        acc_s, acc_h, dec_f = carry
        go = g * 16
        d = dur_v[pl.ds(go, 16)]
        evv = ev_v[pl.ds(go, 16)]
        is_ev = evv != 0

        a = d * inv_step
        c = a.astype(jnp.int32)
        bb = jnp.clip(c - 1, 0, K - 3)
        p = bb
        for t in range(4):
            ec = plsc.load_gather(edges_v, [jnp.minimum(bb + t, K)])
            p = p + jnp.where(ec < d, 1, 0).astype(jnp.int32)
        idx = jnp.clip(p - 1, 0, K - 1)

        def chunk_body(jj, ch):
            (e_s, m_s, e_h, m_h, prev_x, s_prev, dmin) = ch
            j0 = jj * UNROLL
            for dj in range(UNROLL):
                j = j0 + dj
                x = pred_v[j, pl.ds(go, 16)]
                dmin = jnp.minimum(dmin, prev_x - x)
                prev_x = x
                m_lt = j < idx
                t_h = jnp.clip(1.0 - x, EPS, 1.0 - EPS)
                t_h = jnp.where(m_lt, t_h, 1.0)
                tb = plsc.bitcast(t_h, jnp.int32)
                e_h = e_h + (tb >> 23)
                m_h = m_h * plsc.bitcast((tb & MASK23) | ONEBITS, jnp.float32)
                s = jnp.maximum(x, EPS)
                t_s = jnp.minimum(s / s_prev, 1.0 - EPS)
                s_prev = s
                t_s = jnp.where(m_lt, t_s, 1.0)
                tb = plsc.bitcast(t_s, jnp.int32)
                e_s = e_s + (tb >> 23)
                m_s = m_s * plsc.bitcast((tb & MASK23) | ONEBITS, jnp.float32)
            mb = plsc.bitcast(m_s, jnp.int32)
            e_s = e_s + (mb >> 23)
            m_s = plsc.bitcast((mb & MASK23) | ONEBITS, jnp.float32)
            mb = plsc.bitcast(m_h, jnp.int32)
            e_h = e_h + (mb >> 23)
            m_h = plsc.bitcast((mb & MASK23) | ONEBITS, jnp.float32)
            return (e_s, m_s, e_h, m_h, prev_x, s_prev, dmin)

        zi = lanes * 0
        zf = zi.astype(jnp.float32)
        init = (zi, zf + 1.0, zi, zf + 1.0, zf + 3e38, zf + 1.0, zf + 3e38)
        (e_s, m_s, e_h, m_h, _, _, dmin) = lax.fori_loop(
            0, NCHUNK, chunk_body, init)
        dec_f = jnp.minimum(dec_f, jnp.where(dmin >= -1e-6, 1.0, 0.0))

        cols = go + lanes
        x_at = plsc.load_gather(pred_v, [idx, cols])
        x_pv = plsc.load_gather(pred_v, [jnp.maximum(idx - 1, 0), cols])
        h_h_at = jnp.clip(x_at, EPS, 1.0 - EPS)
        s_at = jnp.clip(x_at, EPS, 1.0)
        s_pv = jnp.where(idx == 0, 1.0, jnp.clip(x_pv, EPS, 1.0))
        h_s_at = jnp.clip(1.0 - s_at / s_pv, EPS, 1.0 - EPS)

        ebias = 127 * (K + NCHUNK + 2)
        tail_s = jnp.where(is_ev, h_s_at, 1.0 - h_s_at)
        tb = plsc.bitcast(tail_s, jnp.int32)
        e_s = e_s + (tb >> 23)
        mm = m_s * plsc.bitcast((tb & MASK23) | ONEBITS, jnp.float32)
        mb = plsc.bitcast(mm, jnp.int32)
        e_s = e_s + (mb >> 23)
        m_c = plsc.bitcast((mb & MASK23) | ONEBITS, jnp.float32)
        ll_s = ((e_s - ebias).astype(jnp.float32) + _log2_mant(m_c)) * LN2
        fin_s = (ll_s > -1e30) & (ll_s < 1e30)
        acc_s = acc_s + jnp.where(fin_s, ll_s, -1e6)

        tail_h = jnp.where(is_ev, h_h_at, 1.0 - h_h_at)
        tb = plsc.bitcast(tail_h, jnp.int32)
        e_h = e_h + (tb >> 23)
        mm = m_h * plsc.bitcast((tb & MASK23) | ONEBITS, jnp.float32)
        mb = plsc.bitcast(mm, jnp.int32)
        e_h = e_h + (mb >> 23)
        m_c = plsc.bitcast((mb & MASK23) | ONEBITS, jnp.float32)
        ll_h = ((e_h - ebias).astype(jnp.float32) + _log2_mant(m_c)) * LN2
        fin_h = (ll_h > -1e30) & (ll_h < 1e30)
        acc_h = acc_h + jnp.where(fin_h, ll_h, -1e6)

        return (acc_s, acc_h, dec_f)

    zf = lanes.astype(jnp.float32) * 0.0
    carry = (zf, zf, zf + 1.0)
    cp0.wait()
    carry = lax.fori_loop(0, NG // 2, group_body, carry)
    cp1.wait()
    carry = lax.fori_loop(NG // 2, NG, group_body, carry)
    acc_s, acc_h, dec_f = carry

    stage_v[pl.ds(0, 16)] = acc_s
    stage_v[pl.ds(16, 16)] = acc_h
    stage_v[pl.ds(32, 16)] = dec_f
    stage_v[pl.ds(48, 16)] = dec_f
    pltpu.sync_copy(stage_v, out_hbm.at[wid])


def _tc_body(edges_ref, pred_ref, dur_ref, ev_ref, out_ref):
    for sub in range(LTC // 128):
        _tc_sub(edges_ref, pred_ref, dur_ref, ev_ref, out_ref, sub)


def _tc_sub(edges_ref, pred_ref, dur_ref, ev_ref, out_ref, sub):
    d = dur_ref[0, pl.ds(sub, 1), :]
    is_ev = ev_ref[0, pl.ds(sub, 1), :] != 0
    db = jnp.broadcast_to(d, (8, 128))

    cnt = jnp.zeros((8, 128), jnp.int32)
    for c in range(NCHUNK):
        ec = jnp.broadcast_to(edges_ref[:, c:c + 1], (8, 128))
        cnt = cnt + jnp.where(ec < db, 1, 0).astype(jnp.int32)
    p = jnp.sum(cnt, axis=0, keepdims=True)
    idx = jnp.clip(p - 1, 0, K - 1)
    idx_b = jnp.broadcast_to(idx, (8, 128))

    ji0 = lax.broadcasted_iota(jnp.int32, (8, 128), 0)

    zf8 = jnp.zeros((8, 128), jnp.float32)
    zi8 = jnp.zeros((8, 128), jnp.int32)
    at_x, at_pv = zf8, zf8
    m_s8, m_h8 = zf8 + 1.0, zf8 + 1.0
    e_s8, e_h8 = zi8, zi8
    dmin = zf8 + 3e38
    carry = jnp.ones((1, 128), jnp.float32)
    for c in range(NCHUNK):
        x8 = pred_ref[pl.ds(c * 8, 8), pl.ds(sub * 128, 128)]
        prev8 = pltpu.roll(x8, 1, 0)
        prev8 = jnp.where(ji0 == 0, jnp.broadcast_to(carry, (8, 128)), prev8)
        carry = x8[7:8, :]
        dmin = jnp.minimum(dmin, prev8 - x8)
        ji = ji0 + (c * 8)
        m_lt = ji < idx_b
        m_eq = ji == idx_b
        at_x = at_x + jnp.where(m_eq, x8, 0.0)
        at_pv = at_pv + jnp.where(m_eq, prev8, 0.0)
        t_h = jnp.clip(1.0 - x8, EPS, 1.0 - EPS)
        t_h = jnp.where(m_lt, t_h, 1.0)
        tb = lax.bitcast_convert_type(t_h, jnp.int32)
        e_h8 = e_h8 + (tb >> 23)
        m_h8 = m_h8 * lax.bitcast_convert_type(
            (tb & MASK23) | ONEBITS, jnp.float32)
        s = jnp.maximum(x8, EPS)
        sp = jnp.maximum(prev8, EPS)
        t_s = jnp.minimum(s / sp, 1.0 - EPS)
        t_s = jnp.where(m_lt, t_s, 1.0)
        tb = lax.bitcast_convert_type(t_s, jnp.int32)
        e_s8 = e_s8 + (tb >> 23)
        m_s8 = m_s8 * lax.bitcast_convert_type(
            (tb & MASK23) | ONEBITS, jnp.float32)

    x_at = jnp.sum(at_x, axis=0, keepdims=True)
    x_pv = jnp.sum(at_pv, axis=0, keepdims=True)
    dminv = jnp.min(dmin, axis=0, keepdims=True)
    decv = jnp.where(dminv >= -1e-6, 1.0, 0.0)

    h_h_at = jnp.clip(x_at, EPS, 1.0 - EPS)
    tail_h = jnp.where(is_ev, h_h_at, 1.0 - h_h_at)
    s_at = jnp.clip(x_at, EPS, 1.0)
    s_pv = jnp.clip(x_pv, EPS, 1.0)
    h_s_at = jnp.clip(1.0 - s_at / s_pv, EPS, 1.0 - EPS)
    tail_s = jnp.where(is_ev, h_s_at, 1.0 - h_s_at)

    ebias = 127 * (K + 8 + 3)

    def branch_ll(e8, m8, tail):
        mb = lax.bitcast_convert_type(m8, jnp.int32)
        e8 = e8 + (mb >> 23)
        mant8 = lax.bitcast_convert_type((mb & MASK23) | ONEBITS, jnp.float32)
        r = mant8 * pltpu.roll(mant8, 4, 0)
        r = r * pltpu.roll(r, 2, 0)
        r = r * pltpu.roll(r, 1, 0)
        esum = jnp.sum(e8, axis=0, keepdims=True)
        r0 = r[0:1, :]
        rb = lax.bitcast_convert_type(r0, jnp.int32)
        esum = esum + (rb >> 23)
        mc = lax.bitcast_convert_type((rb & MASK23) | ONEBITS, jnp.float32)
        tb2 = lax.bitcast_convert_type(tail, jnp.int32)
        esum = esum + (tb2 >> 23)
        mm = mc * lax.bitcast_convert_type(
            (tb2 & MASK23) | ONEBITS, jnp.float32)
        mb2 = lax.bitcast_convert_type(mm, jnp.int32)
        esum = esum + (mb2 >> 23)
        m_f = lax.bitcast_convert_type((mb2 & MASK23) | ONEBITS, jnp.float32)
        return ((esum - ebias).astype(jnp.float32) + _log2_mant(m_f)) * LN2

    ll_s = branch_ll(e_s8, m_s8, tail_s)
    ll_h = branch_ll(e_h8, m_h8, tail_h)
    ll_s = jnp.where((ll_s > -1e30) & (ll_s < 1e30), ll_s, -1e6)
    ll_h = jnp.where((ll_h > -1e30) & (ll_h < 1e30), ll_h, -1e6)

    z = jnp.zeros((1, 128), jnp.float32)
    out_ref[0, sub] = jnp.concatenate(
        [ll_s, ll_h, decv, z, z, z, z, z], axis=0)


@jax.jit
def kernel(pred_prob, true_time, true_event):
    pred = pred_prob.astype(jnp.float32)
    dur = true_time.astype(jnp.float32).reshape(-1)
    ev = true_event.reshape(-1).astype(jnp.int32)
    max_t = jnp.clip(jnp.max(dur), 1e-6, None)
    edges = jnp.linspace(0.0, max_t, K + 1).astype(jnp.float32)
    edges_pad = jnp.zeros((224,), jnp.float32)
    edges_pad = edges_pad.at[:K + 1].set(edges)
    edges_pad = edges_pad.at[208:].set(jnp.float32(K) / max_t)
    edges_tc = jnp.zeros((8, 128), jnp.float32)
    edges_tc = edges_tc.at[:, :NCHUNK].set(edges[:K].reshape(NCHUNK, 8).T)

    predt = pred.T
    dur3 = dur.reshape(B // LTC, LTC // 128, 128)
    ev3 = ev.reshape(B // LTC, LTC // 128, 128)

    mesh = plsc.VectorSubcoreMesh(core_axis_name="c", subcore_axis_name="s",
                                  num_cores=NC, num_subcores=NS)
    run_sc = pl.kernel(
        _sc_body,
        out_type=jax.ShapeDtypeStruct((NW, 64), jnp.float32),
        mesh=mesh,
        compiler_params=pltpu.CompilerParams(needs_layout_passes=False),
        scratch_types=[
            pltpu.VMEM((K, RPW), jnp.float32),
            pltpu.VMEM((224,), jnp.float32),
            pltpu.VMEM((RPW,), jnp.float32),
            pltpu.VMEM((RPW,), jnp.int32),
            pltpu.VMEM((64,), jnp.float32),
            pltpu.SemaphoreType.DMA,
            pltpu.SemaphoreType.DMA,
        ],
    )
    parts_sc = run_sc(predt, edges_pad, dur, ev)

    blk0 = B_SC // LTC
    parts_tc = pl.pallas_call(
        _tc_body,
        grid=(NB_TC,),
        in_specs=[
            pl.BlockSpec((8, 128), lambda i: (0, 0)),
            pl.BlockSpec((K, LTC), lambda i: (0, i + blk0)),
            pl.BlockSpec((1, LTC // 128, 128), lambda i: (i + blk0, 0, 0)),
            pl.BlockSpec((1, LTC // 128, 128), lambda i: (i + blk0, 0, 0)),
        ],
        out_specs=pl.BlockSpec((1, LTC // 128, 8, 128), lambda i: (i, 0, 0, 0)),
        out_shape=jax.ShapeDtypeStruct((NB_TC, LTC // 128, 8, 128),
                                       jnp.float32),
    )(edges_tc, predt, dur3, ev3)

    sum_s = jnp.sum(parts_sc[:, 0:16]) + jnp.sum(parts_tc[:, :, 0, :])
    sum_h = jnp.sum(parts_sc[:, 16:32]) + jnp.sum(parts_tc[:, :, 1, :])
    dec = jnp.minimum(jnp.min(parts_sc[:, 32:48]),
                      jnp.min(parts_tc[:, :, 2, :]))
    cond = dec > 0.5
    return -jnp.where(cond, sum_s, sum_h) / B
